# baseline (speedup 1.0000x reference)
import functools

import jax
import jax.numpy as jnp
from jax import lax
from jax.experimental import pallas as pl
from jax.experimental.pallas import tpu as pltpu
from jax.experimental.pallas import tpu_sc as plsc

N = 2048
D = 1024
H = 4096
E = 8
EPS = 1e-5

TM = 256
NROWS = 2 * N + E * TM
NCHUNK = NROWS // TM
HB = 1024
NHB = H // HB
DAUG = D + 128

NC_SC = 2
NS_SC = 16
NW = NC_SC * NS_SC
TPW = N // NW
SB = 16


def _prep_body(x_ref, gam_ref, bet_ref, wr_ref,
               xg1_ref, xg2_ref, pos1_ref, pos2_ref, meta_ref, aux_ref):
    x = x_ref[...]
    mu = jnp.mean(x, axis=1, keepdims=True)
    xc = x - mu
    var = jnp.mean(xc * xc, axis=1, keepdims=True)
    xn = xc / jnp.sqrt(var + EPS) * gam_ref[...] + bet_ref[...]

    logits = lax.dot_general(xn, wr_ref[...], (((1,), (1,)), ((), ())),
                             preferred_element_type=jnp.float32)

    ie = lax.broadcasted_iota(jnp.int32, (N, E), 1)
    m1 = jnp.max(logits, axis=1, keepdims=True)
    i1 = jnp.min(jnp.where(logits >= m1, ie, E), axis=1, keepdims=True)
    mask1 = ie == i1
    l2 = jnp.where(mask1, -jnp.inf, logits)
    m2 = jnp.max(l2, axis=1, keepdims=True)
    i2 = jnp.min(jnp.where(l2 >= m2, ie, E), axis=1, keepdims=True)
    mask2 = ie == i2

    g1 = 1.0 / (1.0 + jnp.exp(m2 - m1))
    g2 = 1.0 - g1

    cmat = mask1.astype(jnp.float32) + mask2.astype(jnp.float32)
    counts = jnp.sum(cmat, axis=0, keepdims=True)
    p = jnp.exp(logits - m1)
    p = p / jnp.sum(p, axis=1, keepdims=True)
    meanp = jnp.mean(p, axis=0, keepdims=True)
    aux = jnp.float32(E) * jnp.sum(counts * meanp) / jnp.float32(N)
    aux_ref[...] = jnp.reshape(aux, (1, 1))

    rr = lax.broadcasted_iota(jnp.int32, (N, N), 0)
    cc = lax.broadcasted_iota(jnp.int32, (N, N), 1)
    tri = (cc < rr).astype(jnp.float32)
    cex = lax.dot_general(tri, cmat, (((1,), (0,)), ((), ())),
                          preferred_element_type=jnp.float32)

    padded = jnp.ceil(counts / TM) * TM
    er = lax.broadcasted_iota(jnp.int32, (E, E), 0)
    ec = lax.broadcasted_iota(jnp.int32, (E, E), 1)
    tril_e = (ec < er).astype(jnp.float32)
    aoff = lax.dot_general(padded, tril_e, (((1,), (1,)), ((), ())),
                           preferred_element_type=jnp.float32)

    rank1 = jnp.sum(jnp.where(mask1, cex, 0.0), axis=1, keepdims=True)
    base1 = jnp.sum(jnp.where(mask1, aoff, 0.0), axis=1, keepdims=True)
    pos1_ref[...] = (base1 + rank1).astype(jnp.int32)
    rank2 = jnp.sum(jnp.where(mask2, cex, 0.0), axis=1, keepdims=True)
    base2 = jnp.sum(jnp.where(mask2, aoff, 0.0), axis=1, keepdims=True)
    pos2_ref[...] = (base2 + rank2).astype(jnp.int32)

    pad0 = jnp.zeros((N, DAUG - D - 1), jnp.float32)
    xg1_ref[...] = jnp.concatenate([xn * g1, g1, pad0], axis=1)
    xg2_ref[...] = jnp.concatenate([xn * g2, g2, pad0], axis=1)

    ck = (lax.broadcasted_iota(jnp.int32, (NCHUNK, E), 0) * TM
          ).astype(jnp.float32)
    ends = aoff + padded
    ce = jnp.minimum(jnp.sum((ck >= ends).astype(jnp.int32), axis=1,
                             keepdims=True), E - 1)
    act = jnp.sum(((ck >= aoff) & (ck < aoff + counts)).astype(jnp.int32),
                  axis=1, keepdims=True)
    e_iota = lax.broadcasted_iota(jnp.int32, (1, E), 1)
    lastce = jnp.max(jnp.where(counts > 0, e_iota, 0))
    ce = jnp.where(act == 1, ce, lastce)
    meta_ref[...] = jnp.concatenate([ce, act], axis=1)


def _prep(xf, gamma, beta, Wr):
    return pl.pallas_call(
        _prep_body,
        out_shape=[
            jax.ShapeDtypeStruct((N, DAUG), jnp.float32),
            jax.ShapeDtypeStruct((N, DAUG), jnp.float32),
            jax.ShapeDtypeStruct((N, 1), jnp.int32),
            jax.ShapeDtypeStruct((N, 1), jnp.int32),
            jax.ShapeDtypeStruct((NCHUNK, 2), jnp.int32),
            jax.ShapeDtypeStruct((1, 1), jnp.float32),
        ],
    )(xf, gamma, beta, Wr)


def _scatter_body(xg1_hbm, xg2_hbm, pos1_hbm, pos2_hbm, xs_hbm,
                  rows_v, idxv, sem):
    wid = lax.axis_index("s") * NC_SC + lax.axis_index("c")
    base = wid * TPW
    for k in range(2):
        xg = xg1_hbm if k == 0 else xg2_hbm
        ph = pos1_hbm if k == 0 else pos2_hbm
        pltpu.sync_copy(xg.at[pl.ds(base, TPW)], rows_v)
        pltpu.sync_copy(ph.at[pl.ds(base, TPW)], idxv)
        pltpu.async_copy(rows_v, xs_hbm.at[idxv], sem).wait()


def _scatter(xg1, xg2, pos1, pos2):
    mesh = plsc.VectorSubcoreMesh(core_axis_name="c", subcore_axis_name="s")
    return pl.kernel(
        _scatter_body,
        out_type=jax.ShapeDtypeStruct((NROWS, DAUG), jnp.float32),
        mesh=mesh,
        scratch_types=[
            pltpu.VMEM((TPW, DAUG), jnp.float32),
            pltpu.VMEM((TPW,), jnp.int32),
            pltpu.SemaphoreType.DMA,
        ],
    )(xg1, xg2, pos1, pos2)


def _moe_body(meta_ref, xs_ref, w1_ref, b1_ref, w2_ref, b2_ref,
              out_ref, acc_ref):
    hb = pl.program_id(1)
    c = pl.program_id(0)

    @pl.when(meta_ref[c, 1] == 1)
    def _():
        xb = xs_ref[...]
        xd = xb[:, :D].astype(jnp.bfloat16)
        gcol = xb[:, D:D + 1]
        z = lax.dot_general(xd, w1_ref[0], (((1,), (1,)), ((), ())),
                            preferred_element_type=jnp.float32)
        h = jnp.maximum(z + gcol * b1_ref[0], 0.0).astype(jnp.bfloat16)
        part = lax.dot_general(h, w2_ref[0], (((1,), (1,)), ((), ())),
                               preferred_element_type=jnp.float32)

        @pl.when(hb == 0)
        def _():
            acc_ref[...] = part + gcol * b2_ref[0]

        @pl.when(hb != 0)
        def _():
            acc_ref[...] = acc_ref[...] + part

        @pl.when(hb == NHB - 1)
        def _():
            out_ref[...] = acc_ref[...]


def _experts(meta, xs, w1, b1, w2, b2):
    def hb_of(c, hb, m):
        return jnp.where(m[c, 1] == 1, hb, NHB - 1)

    grid_spec = pltpu.PrefetchScalarGridSpec(
        num_scalar_prefetch=1,
        grid=(NCHUNK, NHB),
        in_specs=[
            pl.BlockSpec((TM, DAUG), lambda c, hb, m: (c, 0)),
            pl.BlockSpec((1, HB, D), lambda c, hb, m: (m[c, 0], hb_of(c, hb, m), 0)),
            pl.BlockSpec((1, 1, HB), lambda c, hb, m: (m[c, 0], 0, hb_of(c, hb, m))),
            pl.BlockSpec((1, D, HB), lambda c, hb, m: (m[c, 0], 0, hb_of(c, hb, m))),
            pl.BlockSpec((1, 1, D), lambda c, hb, m: (m[c, 0], 0, 0)),
        ],
        out_specs=pl.BlockSpec((TM, D), lambda c, hb, m: (c, 0)),
        scratch_shapes=[pltpu.VMEM((TM, D), jnp.float32)],
    )
    return pl.pallas_call(
        _moe_body,
        grid_spec=grid_spec,
        out_shape=jax.ShapeDtypeStruct((NROWS, D), jnp.float32),
    )(meta, xs, w1.astype(jnp.bfloat16), b1.reshape(E, 1, H),
      w2.astype(jnp.bfloat16), b2.reshape(E, 1, D))


def _combine_body(ys_hbm, pos1_hbm, pos2_hbm, xf_hbm, out_hbm,
                  idxv, ya, yb, xo, sem):
    wid = lax.axis_index("s") * NC_SC + lax.axis_index("c")
    base = wid * TPW
    pltpu.sync_copy(pos1_hbm.at[pl.ds(base, TPW)], idxv.at[pl.ds(0, TPW)])
    pltpu.sync_copy(pos2_hbm.at[pl.ds(base, TPW)], idxv.at[pl.ds(TPW, TPW)])

    for sb in range(TPW // SB):
        t0 = base + sb * SB
        pltpu.sync_copy(xf_hbm.at[pl.ds(t0, SB)], xo)
        ia = idxv[pl.ds(sb * SB, SB)]
        ib = idxv[pl.ds(TPW + sb * SB, SB)]
        cp_a = pltpu.async_copy(ys_hbm.at[ia], ya, sem)
        cp_b = pltpu.async_copy(ys_hbm.at[ib], yb, sem)
        cp_a.wait()
        cp_b.wait()

        def rowcol(i, carry):
            r = i // (D // 64)
            c0 = (i % (D // 64)) * 64
            for u in range(4):
                cl = c0 + u * 16
                xo[r, pl.ds(cl, 16)] = (xo[r, pl.ds(cl, 16)]
                                        + ya[r, pl.ds(cl, 16)]
                                        + yb[r, pl.ds(cl, 16)])
            return carry

        lax.fori_loop(0, SB * (D // 64), rowcol, 0)
        pltpu.sync_copy(xo, out_hbm.at[pl.ds(t0, SB)])


def _combine(ys, pos1, pos2, xf):
    mesh = plsc.VectorSubcoreMesh(core_axis_name="c", subcore_axis_name="s")
    return pl.kernel(
        _combine_body,
        out_type=jax.ShapeDtypeStruct((N, D), jnp.float32),
        mesh=mesh,
        scratch_types=[
            pltpu.VMEM((2 * TPW,), jnp.int32),
            pltpu.VMEM((SB, D), jnp.float32),
            pltpu.VMEM((SB, D), jnp.float32),
            pltpu.VMEM((SB, D), jnp.float32),
            pltpu.SemaphoreType.DMA,
        ],
    )(ys, pos1, pos2, xf)


def kernel(x, gamma, beta, Wr, w1, b1, w2, b2):
    b, s, d = x.shape
    xf = x.reshape(b * s, d)
    xg1, xg2, pos1, pos2, meta, aux = _prep(
        xf, gamma.reshape(1, D), beta.reshape(1, D), Wr)
    pos1 = pos1.reshape(N)
    pos2 = pos2.reshape(N)
    xs = _scatter(xg1, xg2, pos1, pos2)
    ys = _experts(meta, xs, w1, b1, w2, b2)
    out = _combine(ys, pos1, pos2, xf)
    return out.reshape(b, s, d), aux.reshape(())


# speedup vs baseline: 1.2199x; 1.2199x over previous
import functools

import jax
import jax.numpy as jnp
from jax import lax
from jax.experimental import pallas as pl
from jax.experimental.pallas import tpu as pltpu
from jax.experimental.pallas import tpu_sc as plsc

N = 2048
D = 1024
H = 4096
E = 8
EPS = 1e-5

TM = 256
NROWS = 2 * N + E * TM
NCHUNK = NROWS // TM
HB = 1024
NHB = H // HB
DAUG = D + 128

NC_SC = 2
NS_SC = 16
NW = NC_SC * NS_SC
TPW = N // NW
SB = 16


def _prep_body(x_ref, gam_ref, bet_ref, wr_ref,
               xg1_ref, xg2_ref, pos1_ref, pos2_ref, meta_ref, aux_ref):
    x = x_ref[...]
    mu = jnp.mean(x, axis=1, keepdims=True)
    xc = x - mu
    var = jnp.mean(xc * xc, axis=1, keepdims=True)
    xn = xc / jnp.sqrt(var + EPS) * gam_ref[...] + bet_ref[...]

    logits = lax.dot_general(xn, wr_ref[...], (((1,), (1,)), ((), ())),
                             preferred_element_type=jnp.float32)

    ie = lax.broadcasted_iota(jnp.int32, (N, E), 1)
    m1 = jnp.max(logits, axis=1, keepdims=True)
    i1 = jnp.min(jnp.where(logits >= m1, ie, E), axis=1, keepdims=True)
    mask1 = ie == i1
    l2 = jnp.where(mask1, -jnp.inf, logits)
    m2 = jnp.max(l2, axis=1, keepdims=True)
    i2 = jnp.min(jnp.where(l2 >= m2, ie, E), axis=1, keepdims=True)
    mask2 = ie == i2

    g1 = 1.0 / (1.0 + jnp.exp(m2 - m1))
    g2 = 1.0 - g1

    cmat = mask1.astype(jnp.float32) + mask2.astype(jnp.float32)
    counts = jnp.sum(cmat, axis=0, keepdims=True)
    p = jnp.exp(logits - m1)
    p = p / jnp.sum(p, axis=1, keepdims=True)
    meanp = jnp.mean(p, axis=0, keepdims=True)
    aux = jnp.float32(E) * jnp.sum(counts * meanp) / jnp.float32(N)
    aux_ref[...] = jnp.reshape(aux, (1, 1))

    rr = lax.broadcasted_iota(jnp.int32, (N, N), 0)
    cc = lax.broadcasted_iota(jnp.int32, (N, N), 1)
    tri = (cc < rr).astype(jnp.float32)
    cex = lax.dot_general(tri, cmat, (((1,), (0,)), ((), ())),
                          preferred_element_type=jnp.float32)

    padded = jnp.ceil(counts / TM) * TM
    er = lax.broadcasted_iota(jnp.int32, (E, E), 0)
    ec = lax.broadcasted_iota(jnp.int32, (E, E), 1)
    tril_e = (ec < er).astype(jnp.float32)
    aoff = lax.dot_general(padded, tril_e, (((1,), (1,)), ((), ())),
                           preferred_element_type=jnp.float32)

    rank1 = jnp.sum(jnp.where(mask1, cex, 0.0), axis=1, keepdims=True)
    base1 = jnp.sum(jnp.where(mask1, aoff, 0.0), axis=1, keepdims=True)
    pos1_ref[...] = (base1 + rank1).astype(jnp.int32)
    rank2 = jnp.sum(jnp.where(mask2, cex, 0.0), axis=1, keepdims=True)
    base2 = jnp.sum(jnp.where(mask2, aoff, 0.0), axis=1, keepdims=True)
    pos2_ref[...] = (base2 + rank2).astype(jnp.int32)

    pad0 = jnp.zeros((N, DAUG - D - 1), jnp.float32)
    xg1_ref[...] = jnp.concatenate([xn * g1, g1, pad0], axis=1)
    xg2_ref[...] = jnp.concatenate([xn * g2, g2, pad0], axis=1)

    ck = (lax.broadcasted_iota(jnp.int32, (NCHUNK, E), 0) * TM
          ).astype(jnp.float32)
    ends = aoff + padded
    ce = jnp.minimum(jnp.sum((ck >= ends).astype(jnp.int32), axis=1,
                             keepdims=True), E - 1)
    act = jnp.sum(((ck >= aoff) & (ck < aoff + counts)).astype(jnp.int32),
                  axis=1, keepdims=True)
    e_iota = lax.broadcasted_iota(jnp.int32, (1, E), 1)
    lastce = jnp.max(jnp.where(counts > 0, e_iota, 0))
    ce = jnp.where(act == 1, ce, lastce)
    meta_ref[...] = jnp.concatenate([ce, act], axis=1)


def _prep(xf, gamma, beta, Wr):
    return pl.pallas_call(
        _prep_body,
        out_shape=[
            jax.ShapeDtypeStruct((N, DAUG), jnp.float32),
            jax.ShapeDtypeStruct((N, DAUG), jnp.float32),
            jax.ShapeDtypeStruct((N, 1), jnp.int32),
            jax.ShapeDtypeStruct((N, 1), jnp.int32),
            jax.ShapeDtypeStruct((NCHUNK, 2), jnp.int32),
            jax.ShapeDtypeStruct((1, 1), jnp.float32),
        ],
    )(xf, gamma, beta, Wr)


def _scatter_body(xg1_hbm, xg2_hbm, pos1_hbm, pos2_hbm, xs_hbm,
                  rows_v, idxv, sem):
    wid = lax.axis_index("s") * NC_SC + lax.axis_index("c")
    base = wid * TPW
    for k in range(2):
        xg = xg1_hbm if k == 0 else xg2_hbm
        ph = pos1_hbm if k == 0 else pos2_hbm
        pltpu.sync_copy(xg.at[pl.ds(base, TPW)], rows_v)
        pltpu.sync_copy(ph.at[pl.ds(base, TPW)], idxv)
        pltpu.async_copy(rows_v, xs_hbm.at[idxv], sem).wait()


def _scatter(xg1, xg2, pos1, pos2):
    mesh = plsc.VectorSubcoreMesh(core_axis_name="c", subcore_axis_name="s")
    return pl.kernel(
        _scatter_body,
        out_type=jax.ShapeDtypeStruct((NROWS, DAUG), jnp.float32),
        mesh=mesh,
        scratch_types=[
            pltpu.VMEM((TPW, DAUG), jnp.float32),
            pltpu.VMEM((TPW,), jnp.int32),
            pltpu.SemaphoreType.DMA,
        ],
    )(xg1, xg2, pos1, pos2)


def _moe_body(meta_ref, xs_ref, w1_ref, b1_ref, w2_ref, b2_ref,
              out_ref, acc_ref):
    hb = pl.program_id(1)
    c = pl.program_id(0)

    @pl.when(meta_ref[c, 1] == 1)
    def _():
        xb = xs_ref[...]
        xd = xb[:, :D].astype(jnp.bfloat16)
        gcol = xb[:, D:D + 1]
        z = lax.dot_general(xd, w1_ref[0].astype(jnp.bfloat16),
                            (((1,), (1,)), ((), ())),
                            preferred_element_type=jnp.float32)
        h = jnp.maximum(z + gcol * b1_ref[0], 0.0).astype(jnp.bfloat16)
        part = lax.dot_general(h, w2_ref[0].astype(jnp.bfloat16),
                               (((1,), (1,)), ((), ())),
                               preferred_element_type=jnp.float32)

        @pl.when(hb == 0)
        def _():
            acc_ref[...] = part + gcol * b2_ref[0]

        @pl.when(hb != 0)
        def _():
            acc_ref[...] = acc_ref[...] + part

        @pl.when(hb == NHB - 1)
        def _():
            out_ref[...] = acc_ref[...]


def _experts(meta, xs, w1, b1, w2, b2):
    def hb_of(c, hb, m):
        return jnp.where(m[c, 1] == 1, hb, NHB - 1)

    grid_spec = pltpu.PrefetchScalarGridSpec(
        num_scalar_prefetch=1,
        grid=(NCHUNK, NHB),
        in_specs=[
            pl.BlockSpec((TM, DAUG), lambda c, hb, m: (c, 0)),
            pl.BlockSpec((1, HB, D), lambda c, hb, m: (m[c, 0], hb_of(c, hb, m), 0)),
            pl.BlockSpec((1, 1, HB), lambda c, hb, m: (m[c, 0], 0, hb_of(c, hb, m))),
            pl.BlockSpec((1, D, HB), lambda c, hb, m: (m[c, 0], 0, hb_of(c, hb, m))),
            pl.BlockSpec((1, 1, D), lambda c, hb, m: (m[c, 0], 0, 0)),
        ],
        out_specs=pl.BlockSpec((TM, D), lambda c, hb, m: (c, 0)),
        scratch_shapes=[pltpu.VMEM((TM, D), jnp.float32)],
    )
    return pl.pallas_call(
        _moe_body,
        grid_spec=grid_spec,
        out_shape=jax.ShapeDtypeStruct((NROWS, D), jnp.float32),
    )(meta, xs, w1, b1.reshape(E, 1, H), w2, b2.reshape(E, 1, D))


def _combine_body(ys_hbm, pos1_hbm, pos2_hbm, xf_hbm, out_hbm,
                  idxv, ya, yb, xo, sem):
    wid = lax.axis_index("s") * NC_SC + lax.axis_index("c")
    base = wid * TPW
    pltpu.sync_copy(pos1_hbm.at[pl.ds(base, TPW)], idxv.at[pl.ds(0, TPW)])
    pltpu.sync_copy(pos2_hbm.at[pl.ds(base, TPW)], idxv.at[pl.ds(TPW, TPW)])

    for sb in range(TPW // SB):
        t0 = base + sb * SB
        pltpu.sync_copy(xf_hbm.at[pl.ds(t0, SB)], xo)
        ia = idxv[pl.ds(sb * SB, SB)]
        ib = idxv[pl.ds(TPW + sb * SB, SB)]
        cp_a = pltpu.async_copy(ys_hbm.at[ia], ya, sem)
        cp_b = pltpu.async_copy(ys_hbm.at[ib], yb, sem)
        cp_a.wait()
        cp_b.wait()

        def rowcol(i, carry):
            r = i // (D // 64)
            c0 = (i % (D // 64)) * 64
            for u in range(4):
                cl = c0 + u * 16
                xo[r, pl.ds(cl, 16)] = (xo[r, pl.ds(cl, 16)]
                                        + ya[r, pl.ds(cl, 16)]
                                        + yb[r, pl.ds(cl, 16)])
            return carry

        lax.fori_loop(0, SB * (D // 64), rowcol, 0)
        pltpu.sync_copy(xo, out_hbm.at[pl.ds(t0, SB)])


def _combine(ys, pos1, pos2, xf):
    mesh = plsc.VectorSubcoreMesh(core_axis_name="c", subcore_axis_name="s")
    return pl.kernel(
        _combine_body,
        out_type=jax.ShapeDtypeStruct((N, D), jnp.float32),
        mesh=mesh,
        scratch_types=[
            pltpu.VMEM((2 * TPW,), jnp.int32),
            pltpu.VMEM((SB, D), jnp.float32),
            pltpu.VMEM((SB, D), jnp.float32),
            pltpu.VMEM((SB, D), jnp.float32),
            pltpu.SemaphoreType.DMA,
        ],
    )(ys, pos1, pos2, xf)


def kernel(x, gamma, beta, Wr, w1, b1, w2, b2):
    b, s, d = x.shape
    xf = x.reshape(b * s, d)
    xg1, xg2, pos1, pos2, meta, aux = _prep(
        xf, gamma.reshape(1, D), beta.reshape(1, D), Wr)
    pos1 = pos1.reshape(N)
    pos2 = pos2.reshape(N)
    xs = _scatter(xg1, xg2, pos1, pos2)
    ys = _experts(meta, xs, w1, b1, w2, b2)
    out = _combine(ys, pos1, pos2, xf)
    return out.reshape(b, s, d), aux.reshape(())


# speedup vs baseline: 1.3151x; 1.0781x over previous
import functools

import jax
import jax.numpy as jnp
from jax import lax
from jax.experimental import pallas as pl
from jax.experimental.pallas import tpu as pltpu
from jax.experimental.pallas import tpu_sc as plsc

N = 2048
D = 1024
H = 4096
E = 8
EPS = 1e-5

TM = 256
NROWS = 2 * N + E * TM
NCHUNK = NROWS // TM
HB = 2048
NHB = H // HB
DAUG = D + 128

NC_SC = 2
NS_SC = 16
NW = NC_SC * NS_SC
TPW = N // NW
SB = 16


def _prep_body(x_ref, gam_ref, bet_ref, wr_ref,
               xg1_ref, xg2_ref, pos1_ref, pos2_ref, meta_ref, aux_ref):
    x = x_ref[...]
    mu = jnp.mean(x, axis=1, keepdims=True)
    xc = x - mu
    var = jnp.mean(xc * xc, axis=1, keepdims=True)
    xn = xc / jnp.sqrt(var + EPS) * gam_ref[...] + bet_ref[...]

    logits = lax.dot_general(xn, wr_ref[...], (((1,), (1,)), ((), ())),
                             preferred_element_type=jnp.float32)

    ie = lax.broadcasted_iota(jnp.int32, (N, E), 1)
    m1 = jnp.max(logits, axis=1, keepdims=True)
    i1 = jnp.min(jnp.where(logits >= m1, ie, E), axis=1, keepdims=True)
    mask1 = ie == i1
    l2 = jnp.where(mask1, -jnp.inf, logits)
    m2 = jnp.max(l2, axis=1, keepdims=True)
    i2 = jnp.min(jnp.where(l2 >= m2, ie, E), axis=1, keepdims=True)
    mask2 = ie == i2

    g1 = 1.0 / (1.0 + jnp.exp(m2 - m1))
    g2 = 1.0 - g1

    cmat = mask1.astype(jnp.float32) + mask2.astype(jnp.float32)
    counts = jnp.sum(cmat, axis=0, keepdims=True)
    p = jnp.exp(logits - m1)
    p = p / jnp.sum(p, axis=1, keepdims=True)
    meanp = jnp.mean(p, axis=0, keepdims=True)
    aux = jnp.float32(E) * jnp.sum(counts * meanp) / jnp.float32(N)
    aux_ref[...] = jnp.reshape(aux, (1, 1))

    rr = lax.broadcasted_iota(jnp.int32, (N, N), 0)
    cc = lax.broadcasted_iota(jnp.int32, (N, N), 1)
    tri = (cc < rr).astype(jnp.float32)
    cex = lax.dot_general(tri, cmat, (((1,), (0,)), ((), ())),
                          preferred_element_type=jnp.float32)

    padded = jnp.ceil(counts / TM) * TM
    er = lax.broadcasted_iota(jnp.int32, (E, E), 0)
    ec = lax.broadcasted_iota(jnp.int32, (E, E), 1)
    tril_e = (ec < er).astype(jnp.float32)
    aoff = lax.dot_general(padded, tril_e, (((1,), (1,)), ((), ())),
                           preferred_element_type=jnp.float32)

    rank1 = jnp.sum(jnp.where(mask1, cex, 0.0), axis=1, keepdims=True)
    base1 = jnp.sum(jnp.where(mask1, aoff, 0.0), axis=1, keepdims=True)
    pos1_ref[...] = (base1 + rank1).astype(jnp.int32)
    rank2 = jnp.sum(jnp.where(mask2, cex, 0.0), axis=1, keepdims=True)
    base2 = jnp.sum(jnp.where(mask2, aoff, 0.0), axis=1, keepdims=True)
    pos2_ref[...] = (base2 + rank2).astype(jnp.int32)

    pad0 = jnp.zeros((N, DAUG - D - 1), jnp.float32)
    xg1_ref[...] = jnp.concatenate([xn * g1, g1, pad0], axis=1)
    xg2_ref[...] = jnp.concatenate([xn * g2, g2, pad0], axis=1)

    ck = (lax.broadcasted_iota(jnp.int32, (NCHUNK, E), 0) * TM
          ).astype(jnp.float32)
    ends = aoff + padded
    ce = jnp.minimum(jnp.sum((ck >= ends).astype(jnp.int32), axis=1,
                             keepdims=True), E - 1)
    act = jnp.sum(((ck >= aoff) & (ck < aoff + counts)).astype(jnp.int32),
                  axis=1, keepdims=True)
    e_iota = lax.broadcasted_iota(jnp.int32, (1, E), 1)
    lastce = jnp.max(jnp.where(counts > 0, e_iota, 0))
    ce = jnp.where(act == 1, ce, lastce)
    meta_ref[...] = jnp.concatenate([ce, act], axis=1)


def _prep(xf, gamma, beta, Wr):
    return pl.pallas_call(
        _prep_body,
        out_shape=[
            jax.ShapeDtypeStruct((N, DAUG), jnp.float32),
            jax.ShapeDtypeStruct((N, DAUG), jnp.float32),
            jax.ShapeDtypeStruct((N, 1), jnp.int32),
            jax.ShapeDtypeStruct((N, 1), jnp.int32),
            jax.ShapeDtypeStruct((NCHUNK, 2), jnp.int32),
            jax.ShapeDtypeStruct((1, 1), jnp.float32),
        ],
    )(xf, gamma, beta, Wr)


def _scatter_body(xg1_hbm, xg2_hbm, pos1_hbm, pos2_hbm, xs_hbm,
                  rows_v, idxv, sem):
    wid = lax.axis_index("s") * NC_SC + lax.axis_index("c")
    base = wid * TPW
    for k in range(2):
        xg = xg1_hbm if k == 0 else xg2_hbm
        ph = pos1_hbm if k == 0 else pos2_hbm
        pltpu.sync_copy(xg.at[pl.ds(base, TPW)], rows_v)
        pltpu.sync_copy(ph.at[pl.ds(base, TPW)], idxv)
        pltpu.async_copy(rows_v, xs_hbm.at[idxv], sem).wait()


def _scatter(xg1, xg2, pos1, pos2):
    mesh = plsc.VectorSubcoreMesh(core_axis_name="c", subcore_axis_name="s")
    return pl.kernel(
        _scatter_body,
        out_type=jax.ShapeDtypeStruct((NROWS, DAUG), jnp.float32),
        mesh=mesh,
        scratch_types=[
            pltpu.VMEM((TPW, DAUG), jnp.float32),
            pltpu.VMEM((TPW,), jnp.int32),
            pltpu.SemaphoreType.DMA,
        ],
    )(xg1, xg2, pos1, pos2)


def _moe_body(meta_ref, xs_ref, w1_ref, b1_ref, w2_ref, b2_ref,
              out_ref, acc_ref):
    hb = pl.program_id(1)
    c = pl.program_id(0)

    @pl.when(meta_ref[c, 1] == 1)
    def _():
        xb = xs_ref[...]
        xd = xb[:, :D].astype(jnp.bfloat16)
        gcol = xb[:, D:D + 1]
        z = lax.dot_general(xd, w1_ref[0].astype(jnp.bfloat16),
                            (((1,), (1,)), ((), ())),
                            preferred_element_type=jnp.float32)
        h = jnp.maximum(z + gcol * b1_ref[0], 0.0).astype(jnp.bfloat16)
        part = lax.dot_general(h, w2_ref[0].astype(jnp.bfloat16),
                               (((1,), (1,)), ((), ())),
                               preferred_element_type=jnp.float32)

        @pl.when(hb == 0)
        def _():
            acc_ref[...] = part + gcol * b2_ref[0]

        @pl.when(hb != 0)
        def _():
            acc_ref[...] = acc_ref[...] + part

        @pl.when(hb == NHB - 1)
        def _():
            out_ref[...] = acc_ref[...]


def _experts(meta, xs, w1, b1, w2, b2):
    def hb_of(c, hb, m):
        zig = jnp.bitwise_xor(hb, jnp.bitwise_and(c, 1))
        return jnp.where(m[c, 1] == 1, zig, NHB - 1)

    grid_spec = pltpu.PrefetchScalarGridSpec(
        num_scalar_prefetch=1,
        grid=(NCHUNK, NHB),
        in_specs=[
            pl.BlockSpec((TM, DAUG), lambda c, hb, m: (c, 0)),
            pl.BlockSpec((1, HB, D), lambda c, hb, m: (m[c, 0], hb_of(c, hb, m), 0)),
            pl.BlockSpec((1, 1, HB), lambda c, hb, m: (m[c, 0], 0, hb_of(c, hb, m))),
            pl.BlockSpec((1, D, HB), lambda c, hb, m: (m[c, 0], 0, hb_of(c, hb, m))),
            pl.BlockSpec((1, 1, D), lambda c, hb, m: (m[c, 0], 0, 0)),
        ],
        out_specs=pl.BlockSpec((TM, D), lambda c, hb, m: (c, 0)),
        scratch_shapes=[pltpu.VMEM((TM, D), jnp.float32)],
    )
    return pl.pallas_call(
        _moe_body,
        grid_spec=grid_spec,
        out_shape=jax.ShapeDtypeStruct((NROWS, D), jnp.float32),
    )(meta, xs, w1, b1.reshape(E, 1, H), w2, b2.reshape(E, 1, D))


def _combine_body(ys_hbm, pos1_hbm, pos2_hbm, xf_hbm, out_hbm,
                  idxv, ya, yb, xo, sem):
    wid = lax.axis_index("s") * NC_SC + lax.axis_index("c")
    base = wid * TPW
    pltpu.sync_copy(pos1_hbm.at[pl.ds(base, TPW)], idxv.at[pl.ds(0, TPW)])
    pltpu.sync_copy(pos2_hbm.at[pl.ds(base, TPW)], idxv.at[pl.ds(TPW, TPW)])

    for sb in range(TPW // SB):
        t0 = base + sb * SB
        pltpu.sync_copy(xf_hbm.at[pl.ds(t0, SB)], xo)
        ia = idxv[pl.ds(sb * SB, SB)]
        ib = idxv[pl.ds(TPW + sb * SB, SB)]
        cp_a = pltpu.async_copy(ys_hbm.at[ia], ya, sem)
        cp_b = pltpu.async_copy(ys_hbm.at[ib], yb, sem)
        cp_a.wait()
        cp_b.wait()

        def rowcol(i, carry):
            r = i // (D // 64)
            c0 = (i % (D // 64)) * 64
            for u in range(4):
                cl = c0 + u * 16
                xo[r, pl.ds(cl, 16)] = (xo[r, pl.ds(cl, 16)]
                                        + ya[r, pl.ds(cl, 16)]
                                        + yb[r, pl.ds(cl, 16)])
            return carry

        lax.fori_loop(0, SB * (D // 64), rowcol, 0)
        pltpu.sync_copy(xo, out_hbm.at[pl.ds(t0, SB)])


def _combine(ys, pos1, pos2, xf):
    mesh = plsc.VectorSubcoreMesh(core_axis_name="c", subcore_axis_name="s")
    return pl.kernel(
        _combine_body,
        out_type=jax.ShapeDtypeStruct((N, D), jnp.float32),
        mesh=mesh,
        scratch_types=[
            pltpu.VMEM((2 * TPW,), jnp.int32),
            pltpu.VMEM((SB, D), jnp.float32),
            pltpu.VMEM((SB, D), jnp.float32),
            pltpu.VMEM((SB, D), jnp.float32),
            pltpu.SemaphoreType.DMA,
        ],
    )(ys, pos1, pos2, xf)


def kernel(x, gamma, beta, Wr, w1, b1, w2, b2):
    b, s, d = x.shape
    xf = x.reshape(b * s, d)
    xg1, xg2, pos1, pos2, meta, aux = _prep(
        xf, gamma.reshape(1, D), beta.reshape(1, D), Wr)
    pos1 = pos1.reshape(N)
    pos2 = pos2.reshape(N)
    xs = _scatter(xg1, xg2, pos1, pos2)
    ys = _experts(meta, xs, w1, b1, w2, b2)
    out = _combine(ys, pos1, pos2, xf)
    return out.reshape(b, s, d), aux.reshape(())


# speedup vs baseline: 1.4419x; 1.0964x over previous
import functools

import jax
import jax.numpy as jnp
from jax import lax
from jax.experimental import pallas as pl
from jax.experimental.pallas import tpu as pltpu
from jax.experimental.pallas import tpu_sc as plsc

N = 2048
D = 1024
H = 4096
E = 8
EPS = 1e-5

TM = 256
NROWS = 2 * N + E * TM
NCHUNK = NROWS // TM
HB = 2048
NHB = H // HB
DAUG = D + 128

NC_SC = 2
NS_SC = 16
NW = NC_SC * NS_SC
TPW = N // NW
SB = 16


def _prep_body(x_ref, gam_ref, bet_ref, wr_ref,
               xg1_ref, xg2_ref, pos1_ref, pos2_ref, meta_ref, aux_ref):
    x = x_ref[...]
    mu = jnp.mean(x, axis=1, keepdims=True)
    xc = x - mu
    var = jnp.mean(xc * xc, axis=1, keepdims=True)
    xn = xc / jnp.sqrt(var + EPS) * gam_ref[...] + bet_ref[...]

    logits = lax.dot_general(xn, wr_ref[...], (((1,), (1,)), ((), ())),
                             preferred_element_type=jnp.float32)

    ie = lax.broadcasted_iota(jnp.int32, (N, E), 1)
    m1 = jnp.max(logits, axis=1, keepdims=True)
    i1 = jnp.min(jnp.where(logits >= m1, ie, E), axis=1, keepdims=True)
    mask1 = ie == i1
    l2 = jnp.where(mask1, -jnp.inf, logits)
    m2 = jnp.max(l2, axis=1, keepdims=True)
    i2 = jnp.min(jnp.where(l2 >= m2, ie, E), axis=1, keepdims=True)
    mask2 = ie == i2

    g1 = 1.0 / (1.0 + jnp.exp(m2 - m1))
    g2 = 1.0 - g1

    cmat = mask1.astype(jnp.float32) + mask2.astype(jnp.float32)
    counts = jnp.sum(cmat, axis=0, keepdims=True)
    p = jnp.exp(logits - m1)
    p = p / jnp.sum(p, axis=1, keepdims=True)
    meanp = jnp.mean(p, axis=0, keepdims=True)
    aux = jnp.float32(E) * jnp.sum(counts * meanp) / jnp.float32(N)
    aux_ref[...] = jnp.reshape(aux, (1, 1))

    rr = lax.broadcasted_iota(jnp.int32, (N, N), 0)
    cc = lax.broadcasted_iota(jnp.int32, (N, N), 1)
    tri = (cc < rr).astype(jnp.float32)
    cex = lax.dot_general(tri, cmat, (((1,), (0,)), ((), ())),
                          preferred_element_type=jnp.float32)

    padded = jnp.ceil(counts / TM) * TM
    er = lax.broadcasted_iota(jnp.int32, (E, E), 0)
    ec = lax.broadcasted_iota(jnp.int32, (E, E), 1)
    tril_e = (ec < er).astype(jnp.float32)
    aoff = lax.dot_general(padded, tril_e, (((1,), (1,)), ((), ())),
                           preferred_element_type=jnp.float32)

    rank1 = jnp.sum(jnp.where(mask1, cex, 0.0), axis=1, keepdims=True)
    base1 = jnp.sum(jnp.where(mask1, aoff, 0.0), axis=1, keepdims=True)
    pos1_ref[...] = (base1 + rank1).astype(jnp.int32)
    rank2 = jnp.sum(jnp.where(mask2, cex, 0.0), axis=1, keepdims=True)
    base2 = jnp.sum(jnp.where(mask2, aoff, 0.0), axis=1, keepdims=True)
    pos2_ref[...] = (base2 + rank2).astype(jnp.int32)

    pad0 = jnp.zeros((N, DAUG - D - 1), jnp.float32)
    xg1_ref[...] = jnp.concatenate([xn * g1, g1, pad0], axis=1)
    xg2_ref[...] = jnp.concatenate([xn * g2, g2, pad0], axis=1)

    ck = (lax.broadcasted_iota(jnp.int32, (NCHUNK, E), 0) * TM
          ).astype(jnp.float32)
    ends = aoff + padded
    ce = jnp.minimum(jnp.sum((ck >= ends).astype(jnp.int32), axis=1,
                             keepdims=True), E - 1)
    act = jnp.sum(((ck >= aoff) & (ck < aoff + counts)).astype(jnp.int32),
                  axis=1, keepdims=True)
    e_iota = lax.broadcasted_iota(jnp.int32, (1, E), 1)
    lastce = jnp.max(jnp.where(counts > 0, e_iota, 0))
    ce = jnp.where(act == 1, ce, lastce)
    meta_ref[...] = jnp.concatenate([ce, act], axis=1)


def _prep(xf, gamma, beta, Wr):
    return pl.pallas_call(
        _prep_body,
        out_shape=[
            jax.ShapeDtypeStruct((N, DAUG), jnp.float32),
            jax.ShapeDtypeStruct((N, DAUG), jnp.float32),
            jax.ShapeDtypeStruct((N, 1), jnp.int32),
            jax.ShapeDtypeStruct((N, 1), jnp.int32),
            jax.ShapeDtypeStruct((NCHUNK, 2), jnp.int32),
            jax.ShapeDtypeStruct((1, 1), jnp.float32),
        ],
    )(xf, gamma, beta, Wr)


def _scatter_body(xg1_hbm, xg2_hbm, pos1_hbm, pos2_hbm, xs_hbm,
                  rows_v, idxv, sem):
    wid = lax.axis_index("s") * NC_SC + lax.axis_index("c")
    base = wid * TPW
    for k in range(2):
        xg = xg1_hbm if k == 0 else xg2_hbm
        ph = pos1_hbm if k == 0 else pos2_hbm
        pltpu.sync_copy(xg.at[pl.ds(base, TPW)], rows_v)
        pltpu.sync_copy(ph.at[pl.ds(base, TPW)], idxv)
        pltpu.async_copy(rows_v, xs_hbm.at[idxv], sem).wait()


def _scatter(xg1, xg2, pos1, pos2):
    mesh = plsc.VectorSubcoreMesh(core_axis_name="c", subcore_axis_name="s")
    return pl.kernel(
        _scatter_body,
        out_type=jax.ShapeDtypeStruct((NROWS, DAUG), jnp.float32),
        mesh=mesh,
        scratch_types=[
            pltpu.VMEM((TPW, DAUG), jnp.float32),
            pltpu.VMEM((TPW,), jnp.int32),
            pltpu.SemaphoreType.DMA,
        ],
    )(xg1, xg2, pos1, pos2)


def _moe_body(meta_ref, xs_ref, w1_ref, b1_ref, w2_ref, b2_ref,
              out_ref, acc_ref):
    hb = pl.program_id(0)
    c = pl.program_id(1)

    @pl.when(meta_ref[c, 1] == 1)
    def _():
        xb = xs_ref[...]
        xd = xb[:, :D].astype(jnp.bfloat16)
        gcol = xb[:, D:D + 1]
        z = lax.dot_general(xd, w1_ref[0].astype(jnp.bfloat16),
                            (((1,), (1,)), ((), ())),
                            preferred_element_type=jnp.float32)
        h = jnp.maximum(z + gcol * b1_ref[0], 0.0).astype(jnp.bfloat16)
        part = lax.dot_general(h, w2_ref[0].astype(jnp.bfloat16),
                               (((1,), (1,)), ((), ())),
                               preferred_element_type=jnp.float32)

        @pl.when(hb == 0)
        def _():
            acc_ref[pl.ds(c * TM, TM), :] = (
                part + gcol * b2_ref[0]).astype(jnp.bfloat16)

        @pl.when(hb == NHB - 1)
        def _():
            out_ref[...] = (
                acc_ref[pl.ds(c * TM, TM), :].astype(jnp.float32) + part)


def _experts(meta, xs, w1, b1, w2, b2):
    assert NHB == 2

    grid_spec = pltpu.PrefetchScalarGridSpec(
        num_scalar_prefetch=1,
        grid=(NHB, NCHUNK),
        in_specs=[
            pl.BlockSpec((TM, DAUG),
                         lambda hb, c, m: (jnp.where(m[c, 1] == 1, c, 0), 0)),
            pl.BlockSpec((1, HB, D), lambda hb, c, m: (m[c, 0], hb, 0)),
            pl.BlockSpec((1, 1, HB), lambda hb, c, m: (m[c, 0], 0, hb)),
            pl.BlockSpec((1, D, HB), lambda hb, c, m: (m[c, 0], 0, hb)),
            pl.BlockSpec((1, 1, D), lambda hb, c, m: (m[c, 0], 0, 0)),
        ],
        out_specs=pl.BlockSpec(
            (TM, D), lambda hb, c, m: (jnp.where(hb == NHB - 1, c, 0), 0)),
        scratch_shapes=[pltpu.VMEM((NROWS, D), jnp.bfloat16)],
    )
    return pl.pallas_call(
        _moe_body,
        grid_spec=grid_spec,
        out_shape=jax.ShapeDtypeStruct((NROWS, D), jnp.float32),
    )(meta, xs, w1, b1.reshape(E, 1, H), w2, b2.reshape(E, 1, D))


def _combine_body(ys_hbm, pos1_hbm, pos2_hbm, xf_hbm, out_hbm,
                  idxv, ya, yb, xo, sem):
    wid = lax.axis_index("s") * NC_SC + lax.axis_index("c")
    base = wid * TPW
    pltpu.sync_copy(pos1_hbm.at[pl.ds(base, TPW)], idxv.at[pl.ds(0, TPW)])
    pltpu.sync_copy(pos2_hbm.at[pl.ds(base, TPW)], idxv.at[pl.ds(TPW, TPW)])

    for sb in range(TPW // SB):
        t0 = base + sb * SB
        pltpu.sync_copy(xf_hbm.at[pl.ds(t0, SB)], xo)
        ia = idxv[pl.ds(sb * SB, SB)]
        ib = idxv[pl.ds(TPW + sb * SB, SB)]
        cp_a = pltpu.async_copy(ys_hbm.at[ia], ya, sem)
        cp_b = pltpu.async_copy(ys_hbm.at[ib], yb, sem)
        cp_a.wait()
        cp_b.wait()

        def rowcol(i, carry):
            r = i // (D // 64)
            c0 = (i % (D // 64)) * 64
            for u in range(4):
                cl = c0 + u * 16
                xo[r, pl.ds(cl, 16)] = (xo[r, pl.ds(cl, 16)]
                                        + ya[r, pl.ds(cl, 16)]
                                        + yb[r, pl.ds(cl, 16)])
            return carry

        lax.fori_loop(0, SB * (D // 64), rowcol, 0)
        pltpu.sync_copy(xo, out_hbm.at[pl.ds(t0, SB)])


def _combine(ys, pos1, pos2, xf):
    mesh = plsc.VectorSubcoreMesh(core_axis_name="c", subcore_axis_name="s")
    return pl.kernel(
        _combine_body,
        out_type=jax.ShapeDtypeStruct((N, D), jnp.float32),
        mesh=mesh,
        scratch_types=[
            pltpu.VMEM((2 * TPW,), jnp.int32),
            pltpu.VMEM((SB, D), jnp.float32),
            pltpu.VMEM((SB, D), jnp.float32),
            pltpu.VMEM((SB, D), jnp.float32),
            pltpu.SemaphoreType.DMA,
        ],
    )(ys, pos1, pos2, xf)


def kernel(x, gamma, beta, Wr, w1, b1, w2, b2):
    b, s, d = x.shape
    xf = x.reshape(b * s, d)
    xg1, xg2, pos1, pos2, meta, aux = _prep(
        xf, gamma.reshape(1, D), beta.reshape(1, D), Wr)
    pos1 = pos1.reshape(N)
    pos2 = pos2.reshape(N)
    xs = _scatter(xg1, xg2, pos1, pos2)
    ys = _experts(meta, xs, w1, b1, w2, b2)
    out = _combine(ys, pos1, pos2, xf)
    return out.reshape(b, s, d), aux.reshape(())
